# Initial kernel scaffold; baseline (speedup 1.0000x reference)
import jax
import jax.numpy as jnp
from jax.experimental import pallas as pl


def kernel(h, edge_index, W1, a1, W2, a2):
    raise NotImplementedError("write your pallas kernel here")


# speedup vs baseline: 12.4239x; 12.4239x over previous
import functools

import jax
import jax.numpy as jnp
from jax import lax
from jax.experimental import pallas as pl
from jax.experimental.pallas import tpu as pltpu
from jax.experimental.pallas import tpu_sc as plsc

N = 10000
E = 320000
D = 128
H = 4

CHUNK = 128
NCHUNKS = 2560
K0 = 110
K1 = 50
KPAD = max(K0, K1)
NP = 10240
NPS = NP // 16

_f32 = jnp.float32


def _tc1_body(h_ref, w_ref, a_ref, z_ref, sl_ref, sr_ref, m_ref):
    hb = h_ref[...]
    w = w_ref[0]
    z = lax.dot_general(hb, w, (((1,), (1,)), ((), ())),
                        preferred_element_type=_f32)
    z_ref[0] = z
    al = a_ref[0, 0, :D]
    ar = a_ref[0, 0, D:]
    sl = jnp.sum(z * al[None, :], axis=1)
    sr = jnp.sum(z * ar[None, :], axis=1)
    sl_ref[0, 0] = sl
    sr_ref[0, 0] = sr
    m = jnp.max(sl) + jnp.max(sr)
    m = jnp.where(m > 0, m, 0.01 * m)
    m_ref[0, 0] = jnp.full((16,), m, _f32)


_tc1 = pl.pallas_call(
    _tc1_body,
    grid=(H,),
    in_specs=[
        pl.BlockSpec((N, D), lambda i: (0, 0)),
        pl.BlockSpec((1, D, D), lambda i: (i, 0, 0)),
        pl.BlockSpec((1, 1, 2 * D), lambda i: (i, 0, 0)),
    ],
    out_specs=[
        pl.BlockSpec((1, N, D), lambda i: (i, 0, 0)),
        pl.BlockSpec((1, 1, N), lambda i: (i, 0, 0)),
        pl.BlockSpec((1, 1, N), lambda i: (i, 0, 0)),
        pl.BlockSpec((1, 1, 16), lambda i: (i, 0, 0)),
    ],
    out_shape=[
        jax.ShapeDtypeStruct((H, N, D), _f32),
        jax.ShapeDtypeStruct((H, 1, N), _f32),
        jax.ShapeDtypeStruct((H, 1, N), _f32),
        jax.ShapeDtypeStruct((H, 1, 16), _f32),
    ],
)


def _lane_bcast(v16, lane):
    idx = jnp.full((16, 1), lane, jnp.int32)
    return lax.gather(
        v16, idx,
        lax.GatherDimensionNumbers(offset_dims=(), collapsed_slice_dims=(0,),
                                   start_index_map=(0,)),
        (1,), mode=lax.GatherScatterMode.PROMISE_IN_BOUNDS)


def _make_sc_agg(nh):
    def body(z_hbm, sl_hbm, sr_hbm, m_hbm, pk_hbm, zer_hbm, zef_hbm,
             numer_hbm, denom_hbm,
             pck, srcc, dstc, idxs_v, idxd_v, slb, srb, exv, rows_v, m_v,
             numer_s, denom_s,
             semr0, semr1, semss0, semss1, semt0, semt1):
        c = lax.axis_index("c")
        s = lax.axis_index("s")
        semr = (semr0, semr1)
        semsl = (semss0, semss1)
        semsr = (semt0, semt1)

        nchunks = jnp.where(c == 0, K0, K1)
        start = jnp.where(c == 0, s * K0, 16 * K0 + s * K1)
        npairs = nchunks // 2

        pltpu.sync_copy(m_hbm, m_v)

        def load_idx(tpair):
            pltpu.sync_copy(pk_hbm.at[pl.ds(start + 2 * tpair, 2)], pck)

        def head_body(hd, carry):
            pltpu.sync_copy(zer_hbm, numer_s.at[pl.ds(s * NPS, NPS)])
            pltpu.sync_copy(zef_hbm, denom_s.at[pl.ds(s * NPS, NPS)])
            plsc.subcore_barrier()
            mvec = m_v[hd]
            off = hd * N

            def fire(j, jj, p):
                for q in range(8):
                    w = pck[jj, pl.ds(q * 16, 16)]
                    sv = lax.bitwise_and(w, jnp.int32(0xFFFF))
                    dv = lax.shift_right_logical(w, jnp.int32(16))
                    srcc[p, pl.ds(q * 16, 16)] = sv
                    dstc[p, pl.ds(q * 16, 16)] = dv
                    if nh > 1:
                        idxs_v[p, pl.ds(q * 16, 16)] = sv + off
                        idxd_v[p, pl.ds(q * 16, 16)] = dv + off
                if nh > 1:
                    isrc, idst = idxs_v.at[p], idxd_v.at[p]
                else:
                    isrc, idst = srcc.at[p], dstc.at[p]
                pltpu.async_copy(z_hbm.at[isrc], rows_v.at[p], semr[p])
                pltpu.async_copy(sl_hbm.at[isrc], slb.at[p], semsl[p])
                pltpu.async_copy(sr_hbm.at[idst], srb.at[p], semsr[p])

            def proc(j, p):
                pltpu.make_async_copy(sl_hbm.at[srcc.at[p]], slb.at[p],
                                      semsl[p]).wait()
                pltpu.make_async_copy(sr_hbm.at[dstc.at[p]], srb.at[p],
                                      semsr[p]).wait()
                for q in range(8):
                    sl16 = slb[p, pl.ds(q * 16, 16)]
                    sr16 = srb[p, pl.ds(q * 16, 16)]
                    e = sl16 + sr16
                    e = jnp.where(e > 0, e, 0.01 * e)
                    ex = jnp.exp(e - mvec)
                    eid = ((start + j) * CHUNK + (q * 16)
                           + lax.iota(jnp.int32, 16))
                    ex = jnp.where(eid < E, ex, 0.0)
                    exv[pl.ds(q * 16, 16)] = ex
                pltpu.sync_copy(exv, denom_s.at[dstc.at[p]], add=True)
                pltpu.make_async_copy(z_hbm.at[srcc.at[p]], rows_v.at[p],
                                      semr[p]).wait()
                for g in range(8):
                    ex16 = exv[pl.ds(g * 16, 16)]
                    for l in range(16):
                        b = _lane_bcast(ex16, l)
                        r = g * 16 + l
                        for q in range(8):
                            rows_v[p, r, pl.ds(q * 16, 16)] = (
                                rows_v[p, r, pl.ds(q * 16, 16)] * b)
                pltpu.sync_copy(rows_v.at[p], numer_s.at[dstc.at[p]],
                                add=True)

            load_idx(0)
            fire(0, 0, 0)
            fire(1, 1, 1)

            def pair_body(t, c2):
                j0 = 2 * t
                proc(j0, 0)

                def mid():
                    load_idx(t + 1)
                    fire(j0 + 2, 0, 0)

                pl.when(t < npairs - 1)(mid)
                proc(j0 + 1, 1)
                pl.when(t < npairs - 1)(lambda: fire(j0 + 3, 1, 1))
                return c2

            lax.fori_loop(0, npairs, pair_body, 0)
            plsc.subcore_barrier()
            pltpu.sync_copy(numer_s.at[pl.ds(s * NPS, NPS)],
                            numer_hbm.at[hd, c, pl.ds(s * NPS, NPS)])
            pltpu.sync_copy(denom_s.at[pl.ds(s * NPS, NPS)],
                            denom_hbm.at[hd, c, pl.ds(s * NPS, NPS)])
            plsc.subcore_barrier()
            return carry

        lax.fori_loop(0, nh, head_body, 0)

    return pl.kernel(
        body,
        compiler_params=pltpu.CompilerParams(needs_layout_passes=False),
        out_type=[
            jax.ShapeDtypeStruct((nh, 2, NP, D), _f32),
            jax.ShapeDtypeStruct((nh, 2, NP), _f32),
        ],
        mesh=plsc.VectorSubcoreMesh(core_axis_name="c", subcore_axis_name="s"),
        scratch_types=[
            pltpu.VMEM((2, CHUNK), jnp.int32),
            pltpu.VMEM((2, CHUNK), jnp.int32),
            pltpu.VMEM((2, CHUNK), jnp.int32),
            pltpu.VMEM((2, CHUNK), jnp.int32),
            pltpu.VMEM((2, CHUNK), jnp.int32),
            pltpu.VMEM((2, CHUNK), _f32),
            pltpu.VMEM((2, CHUNK), _f32),
            pltpu.VMEM((CHUNK,), _f32),
            pltpu.VMEM((2, CHUNK, D), _f32),
            pltpu.VMEM((nh, 16), _f32),
            pltpu.VMEM_SHARED((NP, D), _f32),
            pltpu.VMEM_SHARED((NP,), _f32),
            pltpu.SemaphoreType.DMA,
            pltpu.SemaphoreType.DMA,
            pltpu.SemaphoreType.DMA,
            pltpu.SemaphoreType.DMA,
            pltpu.SemaphoreType.DMA,
            pltpu.SemaphoreType.DMA,
        ],
    )


_sc_agg4 = _make_sc_agg(H)
_sc_agg1 = _make_sc_agg(1)


_BN = 1024


def _tc2_body(p0, p1, p2, p3, d0, d1, d2, d3, w_ref, z2_ref):
    i = pl.program_id(0)
    xs = []
    for p_ref, d_ref in ((p0, d0), (p1, d1), (p2, d2), (p3, d3)):
        num = p_ref[0] + p_ref[1]
        den = jnp.maximum(d_ref[0, pl.ds(i * _BN, _BN)]
                          + d_ref[1, pl.ds(i * _BN, _BN)], 1e-16)
        xh = num / den[:, None]
        xs.append(jnp.where(xh > 0, xh, jnp.exp(xh) - 1.0))
    x = jnp.concatenate(xs, axis=1)
    z2_ref[...] = lax.dot_general(x, w_ref[...], (((1,), (1,)), ((), ())),
                                  preferred_element_type=_f32)


_tc2 = pl.pallas_call(
    _tc2_body,
    grid=(NP // _BN,),
    in_specs=[pl.BlockSpec((2, _BN, D), lambda i: (0, i, 0))] * H
    + [pl.BlockSpec((2, NP), lambda i: (0, 0))] * H
    + [pl.BlockSpec((D, H * D), lambda i: (0, 0))],
    out_specs=pl.BlockSpec((_BN, D), lambda i: (i, 0)),
    out_shape=jax.ShapeDtypeStruct((NP, D), _f32),
)


def _tc2b_body(z_ref, a_ref, sl_ref, sr_ref, m_ref):
    z = z_ref[...]
    sl = jnp.sum(z * a_ref[0][None, :], axis=1)
    sr = jnp.sum(z * a_ref[1][None, :], axis=1)
    sl_ref[0] = sl
    sr_ref[0] = sr
    m = jnp.max(sl) + jnp.max(sr)
    m = jnp.where(m > 0, m, 0.01 * m)
    m_ref[0] = jnp.full((16,), m, _f32)


_tc2b = pl.pallas_call(
    _tc2b_body,
    in_specs=[pl.BlockSpec((NP, D), lambda: (0, 0)),
              pl.BlockSpec((2, D), lambda: (0, 0))],
    out_specs=[pl.BlockSpec((1, NP), lambda: (0, 0)),
               pl.BlockSpec((1, NP), lambda: (0, 0)),
               pl.BlockSpec((1, 16), lambda: (0, 0))],
    out_shape=[jax.ShapeDtypeStruct((1, NP), _f32),
               jax.ShapeDtypeStruct((1, NP), _f32),
               jax.ShapeDtypeStruct((1, 16), _f32)],
)


def _tc3_body(p_ref, d_ref, out_ref):
    i = pl.program_id(0)
    num = p_ref[0] + p_ref[1]
    den = jnp.maximum(d_ref[0, pl.ds(i * _BN, _BN)]
                      + d_ref[1, pl.ds(i * _BN, _BN)], 1e-16)
    out_ref[...] = num / den[:, None]


_tc3 = pl.pallas_call(
    _tc3_body,
    grid=(NP // _BN,),
    in_specs=[pl.BlockSpec((2, _BN, D), lambda i: (0, i, 0)),
              pl.BlockSpec((2, NP), lambda i: (0, 0))],
    out_specs=pl.BlockSpec((_BN, D), lambda i: (i, 0)),
    out_shape=jax.ShapeDtypeStruct((NP, D), _f32),
)


@jax.jit
def kernel(h, edge_index, W1, a1, W2, a2):
    src = edge_index[0]
    dst = edge_index[1]
    pad = (NCHUNKS + KPAD) * CHUNK - E
    packed = jnp.bitwise_or(src, jnp.left_shift(dst, 16))
    pkj = jnp.concatenate([packed, jnp.zeros((pad,), jnp.int32)]).reshape(
        NCHUNKS + KPAD, CHUNK)
    zer = jnp.zeros((NPS, D), _f32)
    zef = jnp.zeros((NPS,), _f32)

    Z1, SL1, SR1, M1 = _tc1(h, W1, a1[:, None, :])

    numer1, denom1 = _sc_agg4(Z1.reshape(H * N, D), SL1.reshape(H * N),
                              SR1.reshape(H * N), M1.reshape(H, 16),
                              pkj, zer, zef)

    z2 = _tc2(numer1[0], numer1[1], numer1[2], numer1[3],
              denom1[0], denom1[1], denom1[2], denom1[3], W2)
    sl2, sr2, m2 = _tc2b(z2, a2.reshape(2, D))
    numer2, denom2 = _sc_agg1(z2, sl2[0], sr2[0], m2, pkj, zer, zef)
    return _tc3(numer2[0], denom2[0])[:N]


# speedup vs baseline: 26.0881x; 2.0998x over previous
import functools

import jax
import jax.numpy as jnp
from jax import lax
from jax.experimental import pallas as pl
from jax.experimental.pallas import tpu as pltpu
from jax.experimental.pallas import tpu_sc as plsc

N = 10000
E = 320000
D = 128
H = 4

CHUNK = 112
K0 = 129
K1 = 51
NCHUNKS = 16 * (K0 + K1)
NP = 10240
NPS = NP // 16

_f32 = jnp.float32


def _tc1_body(h_ref, w_ref, a_ref, z_ref, sl_ref, sr_ref, m_ref):
    hb = h_ref[...]
    w = w_ref[0]
    z = lax.dot_general(hb, w, (((1,), (1,)), ((), ())),
                        preferred_element_type=_f32)
    z_ref[0] = z
    al = a_ref[0, 0, :D]
    ar = a_ref[0, 0, D:]
    sl = jnp.sum(z * al[None, :], axis=1)
    sr = jnp.sum(z * ar[None, :], axis=1)
    sl_ref[0, 0] = sl
    sr_ref[0, 0] = sr
    m = jnp.max(sl) + jnp.max(sr)
    m = jnp.where(m > 0, m, 0.01 * m)
    m_ref[0, 0] = jnp.full((16,), m, _f32)


_tc1 = pl.pallas_call(
    _tc1_body,
    grid=(H,),
    in_specs=[
        pl.BlockSpec((N, D), lambda i: (0, 0)),
        pl.BlockSpec((1, D, D), lambda i: (i, 0, 0)),
        pl.BlockSpec((1, 1, 2 * D), lambda i: (i, 0, 0)),
    ],
    out_specs=[
        pl.BlockSpec((1, N, D), lambda i: (i, 0, 0)),
        pl.BlockSpec((1, 1, N), lambda i: (i, 0, 0)),
        pl.BlockSpec((1, 1, N), lambda i: (i, 0, 0)),
        pl.BlockSpec((1, 1, 16), lambda i: (i, 0, 0)),
    ],
    out_shape=[
        jax.ShapeDtypeStruct((H, N, D), _f32),
        jax.ShapeDtypeStruct((H, 1, N), _f32),
        jax.ShapeDtypeStruct((H, 1, N), _f32),
        jax.ShapeDtypeStruct((H, 1, 16), _f32),
    ],
)


def _lane_bcast(v16, lane):
    idx = jnp.full((16, 1), lane, jnp.int32)
    return lax.gather(
        v16, idx,
        lax.GatherDimensionNumbers(offset_dims=(), collapsed_slice_dims=(0,),
                                   start_index_map=(0,)),
        (1,), mode=lax.GatherScatterMode.PROMISE_IN_BOUNDS)


def _make_sc_agg(nh):
    def body(z_hbm, sl_hbm, sr_hbm, m_hbm, pk_hbm, zer_hbm, zef_hbm,
             numer_hbm, denom_hbm,
             pck, srcc, dstc, idxs_v, idxd_v, slb, srb, exv, rows_v, m_v,
             numer_s, denom_s,
             semr0, semr1, semr2, semsl0, semsl1, semsl2,
             semsr0, semsr1, semsr2, semd0, semd1, semd2,
             semsc0, semsc1, semsc2, semi0, semi1, semi2):
        c = lax.axis_index("c")
        s = lax.axis_index("s")
        semr = (semr0, semr1, semr2)
        semsl = (semsl0, semsl1, semsl2)
        semsr = (semsr0, semsr1, semsr2)
        semd = (semd0, semd1, semd2)
        semsc = (semsc0, semsc1, semsc2)
        semi = (semi0, semi1, semi2)

        nchunks = jnp.where(c == 0, K0, K1)
        start = jnp.where(c == 0, s * K0, 16 * K0 + s * K1)
        ntriples = nchunks // 3

        pltpu.sync_copy(m_hbm, m_v)

        def head_body(hd, carry):
            pltpu.sync_copy(zer_hbm, numer_s.at[pl.ds(s * NPS, NPS)])
            pltpu.sync_copy(zef_hbm, denom_s.at[pl.ds(s * NPS, NPS)])
            plsc.subcore_barrier()
            mvec = m_v[hd]
            off = hd * N

            def fire(j, b):
                for q in range(CHUNK // 16):
                    w = pck[b, pl.ds(q * 16, 16)]
                    sv = lax.bitwise_and(w, jnp.int32(0xFFFF))
                    dv = lax.shift_right_logical(w, jnp.int32(16))
                    srcc[b, pl.ds(q * 16, 16)] = sv
                    dstc[b, pl.ds(q * 16, 16)] = dv
                    if nh > 1:
                        idxs_v[b, pl.ds(q * 16, 16)] = sv + off
                        idxd_v[b, pl.ds(q * 16, 16)] = dv + off
                if nh > 1:
                    isrc, idst = idxs_v.at[b], idxd_v.at[b]
                else:
                    isrc, idst = srcc.at[b], dstc.at[b]
                pltpu.async_copy(z_hbm.at[isrc], rows_v.at[b], semr[b])
                pltpu.async_copy(sl_hbm.at[isrc], slb.at[b], semsl[b])
                pltpu.async_copy(sr_hbm.at[idst], srb.at[b], semsr[b])

            def proc(j, b):
                isrc = idxs_v.at[b] if nh > 1 else srcc.at[b]
                idst = idxd_v.at[b] if nh > 1 else dstc.at[b]
                pltpu.make_async_copy(sl_hbm.at[isrc], slb.at[b],
                                      semsl[b]).wait()
                pltpu.make_async_copy(sr_hbm.at[idst], srb.at[b],
                                      semsr[b]).wait()
                for q in range(CHUNK // 16):
                    sl16 = slb[b, pl.ds(q * 16, 16)]
                    sr16 = srb[b, pl.ds(q * 16, 16)]
                    e = sl16 + sr16
                    e = jnp.where(e > 0, e, 0.01 * e)
                    ex = jnp.exp(e - mvec)
                    eid = ((start + j) * CHUNK + (q * 16)
                           + lax.iota(jnp.int32, 16))
                    ex = jnp.where(eid < E, ex, 0.0)
                    exv[b, pl.ds(q * 16, 16)] = ex
                pltpu.async_copy(exv.at[b], denom_s.at[dstc.at[b]], semd[b],
                                 add=True)
                pltpu.make_async_copy(z_hbm.at[isrc], rows_v.at[b],
                                      semr[b]).wait()

                def scale_body(g, cc):
                    gs = pl.multiple_of(g * 16, 16)
                    ex16 = exv[b, pl.ds(gs, 16)]
                    for l in range(16):
                        bc = _lane_bcast(ex16, l)
                        r = g * 16 + l
                        for q in range(8):
                            rows_v[b, r, pl.ds(q * 16, 16)] = (
                                rows_v[b, r, pl.ds(q * 16, 16)] * bc)
                    return cc

                lax.fori_loop(0, CHUNK // 16, scale_body, 0)
                pltpu.async_copy(rows_v.at[b], numer_s.at[dstc.at[b]],
                                 semsc[b], add=True)

            def tail(j, b):
                bp = (b + 2) % 3

                def drain_prev():
                    pltpu.make_async_copy(exv.at[bp],
                                          denom_s.at[dstc.at[bp]],
                                          semd[bp]).wait()
                    pltpu.make_async_copy(rows_v.at[bp],
                                          numer_s.at[dstc.at[bp]],
                                          semsc[bp]).wait()

                pl.when(j >= 1)(drain_prev)

                def next_gather():
                    pl.when(j >= 1)(lambda: pltpu.make_async_copy(
                        pk_hbm.at[pl.ds(0, CHUNK)], pck.at[bp],
                        semi[bp]).wait())
                    fire(j + 2, bp)

                pl.when(j + 2 < nchunks)(next_gather)

                def fire_next_idx():
                    pltpu.async_copy(
                        pk_hbm.at[pl.ds((start + j + 3) * CHUNK, CHUNK)],
                        pck.at[b], semi[b])

                pl.when(j + 3 < nchunks)(fire_next_idx)

            for b0 in range(3):
                pltpu.sync_copy(
                    pk_hbm.at[pl.ds((start + b0) * CHUNK, CHUNK)],
                    pck.at[b0])
            fire(0, 0)
            fire(1, 1)

            def triple_body(u, c2):
                for b in range(3):
                    j = 3 * u + b
                    proc(j, b)
                    tail(j, b)
                return c2

            lax.fori_loop(0, ntriples, triple_body, 0)
            pltpu.make_async_copy(exv.at[2], denom_s.at[dstc.at[2]],
                                  semd[2]).wait()
            pltpu.make_async_copy(rows_v.at[2], numer_s.at[dstc.at[2]],
                                  semsc[2]).wait()
            plsc.subcore_barrier()
            pltpu.sync_copy(numer_s.at[pl.ds(s * NPS, NPS)],
                            numer_hbm.at[hd, c, pl.ds(s * NPS, NPS)])
            pltpu.sync_copy(denom_s.at[pl.ds(s * NPS, NPS)],
                            denom_hbm.at[hd, c, pl.ds(s * NPS, NPS)])
            plsc.subcore_barrier()
            return carry

        lax.fori_loop(0, nh, head_body, 0)

    return pl.kernel(
        body,
        compiler_params=pltpu.CompilerParams(needs_layout_passes=False),
        out_type=[
            jax.ShapeDtypeStruct((nh, 2, NP, D), _f32),
            jax.ShapeDtypeStruct((nh, 2, NP), _f32),
        ],
        mesh=plsc.VectorSubcoreMesh(core_axis_name="c", subcore_axis_name="s"),
        scratch_types=[
            pltpu.VMEM((3, CHUNK), jnp.int32),
            pltpu.VMEM((3, CHUNK), jnp.int32),
            pltpu.VMEM((3, CHUNK), jnp.int32),
            pltpu.VMEM((3, CHUNK), jnp.int32),
            pltpu.VMEM((3, CHUNK), jnp.int32),
            pltpu.VMEM((3, CHUNK), _f32),
            pltpu.VMEM((3, CHUNK), _f32),
            pltpu.VMEM((3, CHUNK), _f32),
            pltpu.VMEM((3, CHUNK, D), _f32),
            pltpu.VMEM((nh, 16), _f32),
            pltpu.VMEM_SHARED((NP, D), _f32),
            pltpu.VMEM_SHARED((NP,), _f32),
        ] + [pltpu.SemaphoreType.DMA] * 18,
    )


_sc_agg4 = _make_sc_agg(H)
_sc_agg1 = _make_sc_agg(1)


_BN = 1024


def _tc2_body(p0, p1, p2, p3, d0, d1, d2, d3, w_ref, z2_ref):
    i = pl.program_id(0)
    xs = []
    for p_ref, d_ref in ((p0, d0), (p1, d1), (p2, d2), (p3, d3)):
        num = p_ref[0] + p_ref[1]
        den = jnp.maximum(d_ref[0, pl.ds(i * _BN, _BN)]
                          + d_ref[1, pl.ds(i * _BN, _BN)], 1e-16)
        xh = num / den[:, None]
        xs.append(jnp.where(xh > 0, xh, jnp.exp(xh) - 1.0))
    x = jnp.concatenate(xs, axis=1)
    z2_ref[...] = lax.dot_general(x, w_ref[...], (((1,), (1,)), ((), ())),
                                  preferred_element_type=_f32)


_tc2 = pl.pallas_call(
    _tc2_body,
    grid=(NP // _BN,),
    in_specs=[pl.BlockSpec((2, _BN, D), lambda i: (0, i, 0))] * H
    + [pl.BlockSpec((2, NP), lambda i: (0, 0))] * H
    + [pl.BlockSpec((D, H * D), lambda i: (0, 0))],
    out_specs=pl.BlockSpec((_BN, D), lambda i: (i, 0)),
    out_shape=jax.ShapeDtypeStruct((NP, D), _f32),
)


def _tc2b_body(z_ref, a_ref, sl_ref, sr_ref, m_ref):
    z = z_ref[...]
    sl = jnp.sum(z * a_ref[0][None, :], axis=1)
    sr = jnp.sum(z * a_ref[1][None, :], axis=1)
    sl_ref[0] = sl
    sr_ref[0] = sr
    m = jnp.max(sl) + jnp.max(sr)
    m = jnp.where(m > 0, m, 0.01 * m)
    m_ref[0] = jnp.full((16,), m, _f32)


_tc2b = pl.pallas_call(
    _tc2b_body,
    in_specs=[pl.BlockSpec((NP, D), lambda: (0, 0)),
              pl.BlockSpec((2, D), lambda: (0, 0))],
    out_specs=[pl.BlockSpec((1, NP), lambda: (0, 0)),
               pl.BlockSpec((1, NP), lambda: (0, 0)),
               pl.BlockSpec((1, 16), lambda: (0, 0))],
    out_shape=[jax.ShapeDtypeStruct((1, NP), _f32),
               jax.ShapeDtypeStruct((1, NP), _f32),
               jax.ShapeDtypeStruct((1, 16), _f32)],
)


def _tc3_body(p_ref, d_ref, out_ref):
    i = pl.program_id(0)
    num = p_ref[0] + p_ref[1]
    den = jnp.maximum(d_ref[0, pl.ds(i * _BN, _BN)]
                      + d_ref[1, pl.ds(i * _BN, _BN)], 1e-16)
    out_ref[...] = num / den[:, None]


_tc3 = pl.pallas_call(
    _tc3_body,
    grid=(NP // _BN,),
    in_specs=[pl.BlockSpec((2, _BN, D), lambda i: (0, i, 0)),
              pl.BlockSpec((2, NP), lambda i: (0, 0))],
    out_specs=pl.BlockSpec((_BN, D), lambda i: (i, 0)),
    out_shape=jax.ShapeDtypeStruct((NP, D), _f32),
)


@jax.jit
def kernel(h, edge_index, W1, a1, W2, a2):
    src = edge_index[0]
    dst = edge_index[1]
    pad = NCHUNKS * CHUNK - E
    packed = jnp.bitwise_or(src, jnp.left_shift(dst, 16))
    pkj = jnp.concatenate([packed, jnp.zeros((pad,), jnp.int32)])
    zer = jnp.zeros((NPS, D), _f32)
    zef = jnp.zeros((NPS,), _f32)

    Z1, SL1, SR1, M1 = _tc1(h, W1, a1[:, None, :])

    numer1, denom1 = _sc_agg4(Z1.reshape(H * N, D), SL1.reshape(H * N),
                              SR1.reshape(H * N), M1.reshape(H, 16),
                              pkj, zer, zef)

    z2 = _tc2(numer1[0], numer1[1], numer1[2], numer1[3],
              denom1[0], denom1[1], denom1[2], denom1[3], W2)
    sl2, sr2, m2 = _tc2b(z2, a2.reshape(2, D))
    numer2, denom2 = _sc_agg1(z2, sl2[0], sr2[0], m2, pkj, zer, zef)
    return _tc3(numer2[0], denom2[0])[:N]


# speedup vs baseline: 27.1416x; 1.0404x over previous
import functools

import jax
import jax.numpy as jnp
from jax import lax
from jax.experimental import pallas as pl
from jax.experimental.pallas import tpu as pltpu
from jax.experimental.pallas import tpu_sc as plsc

N = 10000
E = 320000
D = 128
H = 4

CHUNK = 112
K0 = 141
K1 = 39
NCHUNKS = 16 * (K0 + K1)
NP = 10240
NPS = NP // 16

_f32 = jnp.float32


def _tc1_body(h_ref, w_ref, a_ref, e_ref, z_ref, sl_ref, sr_ref, m_ref,
              pk_ref):
    i = pl.program_id(0)

    @pl.when(i == 0)
    def _pack_edges():
        packed = jnp.bitwise_or(e_ref[0], jnp.left_shift(e_ref[1], 16))
        pk_ref[0, :E] = packed
        pk_ref[0, E:] = jnp.zeros((NCHUNKS * CHUNK - E,), jnp.int32)

    hb = h_ref[...]
    w = w_ref[0]
    z = lax.dot_general(hb, w, (((1,), (1,)), ((), ())),
                        preferred_element_type=_f32)
    z_ref[0] = z
    al = a_ref[0, 0, :D]
    ar = a_ref[0, 0, D:]
    sl = jnp.sum(z * al[None, :], axis=1)
    sr = jnp.sum(z * ar[None, :], axis=1)
    sl_ref[0, 0] = sl
    sr_ref[0, 0] = sr
    m = jnp.max(sl) + jnp.max(sr)
    m_ref[0, 0] = jnp.full((16,), m, _f32)


_tc1 = pl.pallas_call(
    _tc1_body,
    grid=(H,),
    in_specs=[
        pl.BlockSpec((N, D), lambda i: (0, 0)),
        pl.BlockSpec((1, D, D), lambda i: (i, 0, 0)),
        pl.BlockSpec((1, 1, 2 * D), lambda i: (i, 0, 0)),
        pl.BlockSpec((2, E), lambda i: (0, 0)),
    ],
    out_specs=[
        pl.BlockSpec((1, N, D), lambda i: (i, 0, 0)),
        pl.BlockSpec((1, 1, N), lambda i: (i, 0, 0)),
        pl.BlockSpec((1, 1, N), lambda i: (i, 0, 0)),
        pl.BlockSpec((1, 1, 16), lambda i: (i, 0, 0)),
        pl.BlockSpec((1, NCHUNKS * CHUNK), lambda i: (0, 0)),
    ],
    out_shape=[
        jax.ShapeDtypeStruct((H, N, D), _f32),
        jax.ShapeDtypeStruct((H, 1, N), _f32),
        jax.ShapeDtypeStruct((H, 1, N), _f32),
        jax.ShapeDtypeStruct((H, 1, 16), _f32),
        jax.ShapeDtypeStruct((1, NCHUNKS * CHUNK), jnp.int32),
    ],
)


def _lane_bcast(v16, lane):
    idx = jnp.full((16, 1), lane, jnp.int32)
    return lax.gather(
        v16, idx,
        lax.GatherDimensionNumbers(offset_dims=(), collapsed_slice_dims=(0,),
                                   start_index_map=(0,)),
        (1,), mode=lax.GatherScatterMode.PROMISE_IN_BOUNDS)


def _make_sc_agg(nh):
    def body(z_hbm, sl_hbm, sr_hbm, m_hbm, pk_hbm, zer_hbm, zef_hbm,
             numer_hbm, denom_hbm,
             pck, srcc, dstc, idxs_v, idxd_v, slb, srb, exv, rows_v, m_v,
             numer_s, denom_s,
             semr0, semr1, semr2, semsl0, semsl1, semsl2,
             semsr0, semsr1, semsr2, semd0, semd1, semd2,
             semsc0, semsc1, semsc2, semi0, semi1, semi2):
        c = lax.axis_index("c")
        s = lax.axis_index("s")
        semr = (semr0, semr1, semr2)
        semsl = (semsl0, semsl1, semsl2)
        semsr = (semsr0, semsr1, semsr2)
        semd = (semd0, semd1, semd2)
        semsc = (semsc0, semsc1, semsc2)
        semi = (semi0, semi1, semi2)

        nchunks = jnp.where(c == 0, K0, K1)
        start = jnp.where(c == 0, s * K0, 16 * K0 + s * K1)
        ntriples = nchunks // 3

        pltpu.sync_copy(m_hbm, m_v)

        def head_body(hd, carry):
            pltpu.sync_copy(zer_hbm, numer_s.at[pl.ds(s * NPS, NPS)])
            pltpu.sync_copy(zef_hbm, denom_s.at[pl.ds(s * NPS, NPS)])
            plsc.subcore_barrier()
            mraw = m_v[hd]
            mvec = jnp.where(mraw > 0, mraw, 0.01 * mraw)
            off = hd * N

            def fire(j, b):
                for q in range(CHUNK // 16):
                    w = pck[b, pl.ds(q * 16, 16)]
                    sv = lax.bitwise_and(w, jnp.int32(0xFFFF))
                    dv = lax.shift_right_logical(w, jnp.int32(16))
                    srcc[b, pl.ds(q * 16, 16)] = sv
                    dstc[b, pl.ds(q * 16, 16)] = dv
                    if nh > 1:
                        idxs_v[b, pl.ds(q * 16, 16)] = sv + off
                        idxd_v[b, pl.ds(q * 16, 16)] = dv + off
                if nh > 1:
                    isrc, idst = idxs_v.at[b], idxd_v.at[b]
                else:
                    isrc, idst = srcc.at[b], dstc.at[b]
                pltpu.async_copy(z_hbm.at[isrc], rows_v.at[b], semr[b])
                pltpu.async_copy(sl_hbm.at[isrc], slb.at[b], semsl[b])
                pltpu.async_copy(sr_hbm.at[idst], srb.at[b], semsr[b])

            def proc(j, b):
                isrc = idxs_v.at[b] if nh > 1 else srcc.at[b]
                idst = idxd_v.at[b] if nh > 1 else dstc.at[b]
                pltpu.make_async_copy(sl_hbm.at[isrc], slb.at[b],
                                      semsl[b]).wait()
                pltpu.make_async_copy(sr_hbm.at[idst], srb.at[b],
                                      semsr[b]).wait()
                for q in range(CHUNK // 16):
                    sl16 = slb[b, pl.ds(q * 16, 16)]
                    sr16 = srb[b, pl.ds(q * 16, 16)]
                    e = sl16 + sr16
                    e = jnp.where(e > 0, e, 0.01 * e)
                    ex = jnp.exp(e - mvec)
                    eid = ((start + j) * CHUNK + (q * 16)
                           + lax.iota(jnp.int32, 16))
                    ex = jnp.where(eid < E, ex, 0.0)
                    exv[b, pl.ds(q * 16, 16)] = ex
                pltpu.async_copy(exv.at[b], denom_s.at[dstc.at[b]], semd[b],
                                 add=True)
                pltpu.make_async_copy(z_hbm.at[isrc], rows_v.at[b],
                                      semr[b]).wait()

                def scale_body(g, cc):
                    gs = pl.multiple_of(g * 16, 16)
                    ex16 = exv[b, pl.ds(gs, 16)]
                    for l in range(16):
                        bc = _lane_bcast(ex16, l)
                        r = g * 16 + l
                        for q in range(8):
                            rows_v[b, r, pl.ds(q * 16, 16)] = (
                                rows_v[b, r, pl.ds(q * 16, 16)] * bc)
                    return cc

                lax.fori_loop(0, CHUNK // 16, scale_body, 0)
                pltpu.async_copy(rows_v.at[b], numer_s.at[dstc.at[b]],
                                 semsc[b], add=True)

            def tail(j, b):
                bp = (b + 2) % 3

                def drain_prev():
                    pltpu.make_async_copy(exv.at[bp],
                                          denom_s.at[dstc.at[bp]],
                                          semd[bp]).wait()
                    pltpu.make_async_copy(rows_v.at[bp],
                                          numer_s.at[dstc.at[bp]],
                                          semsc[bp]).wait()

                pl.when(j >= 1)(drain_prev)

                def next_gather():
                    pl.when(j >= 1)(lambda: pltpu.make_async_copy(
                        pk_hbm.at[pl.ds(0, CHUNK)], pck.at[bp],
                        semi[bp]).wait())
                    fire(j + 2, bp)

                pl.when(j + 2 < nchunks)(next_gather)

                def fire_next_idx():
                    pltpu.async_copy(
                        pk_hbm.at[pl.ds((start + j + 3) * CHUNK, CHUNK)],
                        pck.at[b], semi[b])

                pl.when(j + 3 < nchunks)(fire_next_idx)

            for b0 in range(3):
                pltpu.sync_copy(
                    pk_hbm.at[pl.ds((start + b0) * CHUNK, CHUNK)],
                    pck.at[b0])
            fire(0, 0)
            fire(1, 1)

            def triple_body(u, c2):
                for b in range(3):
                    j = 3 * u + b
                    proc(j, b)
                    tail(j, b)
                return c2

            lax.fori_loop(0, ntriples, triple_body, 0)
            pltpu.make_async_copy(exv.at[2], denom_s.at[dstc.at[2]],
                                  semd[2]).wait()
            pltpu.make_async_copy(rows_v.at[2], numer_s.at[dstc.at[2]],
                                  semsc[2]).wait()
            plsc.subcore_barrier()
            pltpu.sync_copy(numer_s.at[pl.ds(s * NPS, NPS)],
                            numer_hbm.at[hd, c, pl.ds(s * NPS, NPS)])
            pltpu.sync_copy(denom_s.at[pl.ds(s * NPS, NPS)],
                            denom_hbm.at[hd, c, pl.ds(s * NPS, NPS)])
            plsc.subcore_barrier()
            return carry

        lax.fori_loop(0, nh, head_body, 0)

    return pl.kernel(
        body,
        compiler_params=pltpu.CompilerParams(needs_layout_passes=False),
        out_type=[
            jax.ShapeDtypeStruct((nh, 2, NP, D), _f32),
            jax.ShapeDtypeStruct((nh, 2, NP), _f32),
        ],
        mesh=plsc.VectorSubcoreMesh(core_axis_name="c", subcore_axis_name="s"),
        scratch_types=[
            pltpu.VMEM((3, CHUNK), jnp.int32),
            pltpu.VMEM((3, CHUNK), jnp.int32),
            pltpu.VMEM((3, CHUNK), jnp.int32),
            pltpu.VMEM((3, CHUNK), jnp.int32),
            pltpu.VMEM((3, CHUNK), jnp.int32),
            pltpu.VMEM((3, CHUNK), _f32),
            pltpu.VMEM((3, CHUNK), _f32),
            pltpu.VMEM((3, CHUNK), _f32),
            pltpu.VMEM((3, CHUNK, D), _f32),
            pltpu.VMEM((nh, 16), _f32),
            pltpu.VMEM_SHARED((NP, D), _f32),
            pltpu.VMEM_SHARED((NP,), _f32),
        ] + [pltpu.SemaphoreType.DMA] * 18,
    )


_sc_agg4 = _make_sc_agg(H)
_sc_agg1 = _make_sc_agg(1)


_BN = 1024


def _tc2_body(p0, p1, p2, p3, d0, d1, d2, d3, w_ref, a_ref,
              z2_ref, sl_ref, sr_ref, m_ref):
    i = pl.program_id(0)
    xs = []
    for p_ref, d_ref in ((p0, d0), (p1, d1), (p2, d2), (p3, d3)):
        num = p_ref[0] + p_ref[1]
        den = jnp.maximum(d_ref[0, pl.ds(i * _BN, _BN)]
                          + d_ref[1, pl.ds(i * _BN, _BN)], 1e-16)
        xh = num / den[:, None]
        xs.append(jnp.where(xh > 0, xh, jnp.exp(xh) - 1.0))
    x = jnp.concatenate(xs, axis=1)
    z2 = lax.dot_general(x, w_ref[...], (((1,), (1,)), ((), ())),
                         preferred_element_type=_f32)
    z2_ref[...] = z2
    sl = jnp.sum(z2 * a_ref[0][None, :], axis=1)
    sr = jnp.sum(z2 * a_ref[1][None, :], axis=1)
    sl_ref[0] = sl
    sr_ref[0] = sr
    m = jnp.full((16,), jnp.max(sl) + jnp.max(sr), _f32)

    @pl.when(i == 0)
    def _init_m():
        m_ref[0] = jnp.full((16,), -3e38, _f32)

    m_ref[0] = jnp.maximum(m_ref[0], m)


_tc2 = pl.pallas_call(
    _tc2_body,
    grid=(NP // _BN,),
    in_specs=[pl.BlockSpec((2, _BN, D), lambda i: (0, i, 0))] * H
    + [pl.BlockSpec((2, NP), lambda i: (0, 0))] * H
    + [pl.BlockSpec((D, H * D), lambda i: (0, 0)),
       pl.BlockSpec((2, D), lambda i: (0, 0))],
    out_specs=[pl.BlockSpec((_BN, D), lambda i: (i, 0)),
               pl.BlockSpec((1, _BN), lambda i: (0, i)),
               pl.BlockSpec((1, _BN), lambda i: (0, i)),
               pl.BlockSpec((1, 16), lambda i: (0, 0))],
    out_shape=[jax.ShapeDtypeStruct((NP, D), _f32),
               jax.ShapeDtypeStruct((1, NP), _f32),
               jax.ShapeDtypeStruct((1, NP), _f32),
               jax.ShapeDtypeStruct((1, 16), _f32)],
)


def _tc3_body(p_ref, d_ref, out_ref):
    i = pl.program_id(0)
    num = p_ref[0] + p_ref[1]
    den = jnp.maximum(d_ref[0, pl.ds(i * _BN, _BN)]
                      + d_ref[1, pl.ds(i * _BN, _BN)], 1e-16)
    out_ref[...] = num / den[:, None]


_tc3 = pl.pallas_call(
    _tc3_body,
    grid=(NP // _BN,),
    in_specs=[pl.BlockSpec((2, _BN, D), lambda i: (0, i, 0)),
              pl.BlockSpec((2, NP), lambda i: (0, 0))],
    out_specs=pl.BlockSpec((_BN, D), lambda i: (i, 0)),
    out_shape=jax.ShapeDtypeStruct((NP, D), _f32),
)


@jax.jit
def kernel(h, edge_index, W1, a1, W2, a2):
    src = edge_index[0]
    dst = edge_index[1]
    zer = jnp.zeros((NPS, D), _f32)
    zef = jnp.zeros((NPS,), _f32)
    Z1, SL1, SR1, M1, pkj2 = _tc1(h, W1, a1[:, None, :], edge_index)
    pkj = pkj2[0]

    numer1, denom1 = _sc_agg4(Z1.reshape(H * N, D), SL1.reshape(H * N),
                              SR1.reshape(H * N), M1.reshape(H, 16),
                              pkj, zer, zef)

    z2, sl2, sr2, m2 = _tc2(numer1[0], numer1[1], numer1[2], numer1[3],
                            denom1[0], denom1[1], denom1[2], denom1[3],
                            W2, a2.reshape(2, D))
    numer2, denom2 = _sc_agg1(z2, sl2[0], sr2[0], m2, pkj, zer, zef)
    return _tc3(numer2[0], denom2[0])[:N]


# speedup vs baseline: 27.1477x; 1.0002x over previous
import jax
import jax.numpy as jnp
from jax import lax
from jax.experimental import pallas as pl
from jax.experimental.pallas import tpu as pltpu
from jax.experimental.pallas import tpu_sc as plsc

N = 10000
E = 320000
D = 128
H = 4

CHUNK = 112
K0 = 141
K1 = 39
NCHUNKS = 16 * (K0 + K1)
NP = 10240
NPS = NP // 16

_f32 = jnp.float32


def _tc1_body(h_ref, w_ref, a_ref, e_ref, z_ref, sl_ref, sr_ref, m_ref,
              pk_ref):
    i = pl.program_id(0)

    @pl.when(i == 0)
    def _pack_edges():
        packed = jnp.bitwise_or(e_ref[0], jnp.left_shift(e_ref[1], 16))
        pk_ref[0, :E] = packed
        pk_ref[0, E:] = jnp.zeros((NCHUNKS * CHUNK - E,), jnp.int32)

    hb = h_ref[...]
    w = w_ref[0]
    z = lax.dot_general(hb, w, (((1,), (1,)), ((), ())),
                        preferred_element_type=_f32)
    z_ref[0] = z
    al = a_ref[0, 0, :D]
    ar = a_ref[0, 0, D:]
    sl = jnp.sum(z * al[None, :], axis=1)
    sr = jnp.sum(z * ar[None, :], axis=1)
    sl_ref[0, 0] = sl
    sr_ref[0, 0] = sr
    m = jnp.max(sl) + jnp.max(sr)
    m_ref[0, 0] = jnp.full((16,), m, _f32)


_tc1 = pl.pallas_call(
    _tc1_body,
    grid=(H,),
    in_specs=[
        pl.BlockSpec((N, D), lambda i: (0, 0)),
        pl.BlockSpec((1, D, D), lambda i: (i, 0, 0)),
        pl.BlockSpec((1, 1, 2 * D), lambda i: (i, 0, 0)),
        pl.BlockSpec((2, E), lambda i: (0, 0)),
    ],
    out_specs=[
        pl.BlockSpec((1, N, D), lambda i: (i, 0, 0)),
        pl.BlockSpec((1, 1, N), lambda i: (i, 0, 0)),
        pl.BlockSpec((1, 1, N), lambda i: (i, 0, 0)),
        pl.BlockSpec((1, 1, 16), lambda i: (i, 0, 0)),
        pl.BlockSpec((1, NCHUNKS * CHUNK), lambda i: (0, 0)),
    ],
    out_shape=[
        jax.ShapeDtypeStruct((H, N, D), _f32),
        jax.ShapeDtypeStruct((H, 1, N), _f32),
        jax.ShapeDtypeStruct((H, 1, N), _f32),
        jax.ShapeDtypeStruct((H, 1, 16), _f32),
        jax.ShapeDtypeStruct((1, NCHUNKS * CHUNK), jnp.int32),
    ],
)


def _lane_bcast(v16, lane):
    idx = jnp.full((16, 1), lane, jnp.int32)
    return lax.gather(
        v16, idx,
        lax.GatherDimensionNumbers(offset_dims=(), collapsed_slice_dims=(0,),
                                   start_index_map=(0,)),
        (1,), mode=lax.GatherScatterMode.PROMISE_IN_BOUNDS)


def _make_sc_agg(nh):
    def body(z_hbm, sl_hbm, sr_hbm, m_hbm, pk_hbm, zer_hbm, zef_hbm,
             numer_hbm, denom_hbm,
             pck, srcc, dstc, idxs_v, idxd_v, slb, srb, exv, rows_v, m_v,
             numer_s, denom_s,
             semr0, semr1, semr2, semsl0, semsl1, semsl2,
             semsr0, semsr1, semsr2, semd0, semd1, semd2,
             semsc0, semsc1, semsc2, semi0, semi1, semi2):
        c = lax.axis_index("c")
        s = lax.axis_index("s")
        semr = (semr0, semr1, semr2)
        semsl = (semsl0, semsl1, semsl2)
        semsr = (semsr0, semsr1, semsr2)
        semd = (semd0, semd1, semd2)
        semsc = (semsc0, semsc1, semsc2)
        semi = (semi0, semi1, semi2)

        nchunks = jnp.where(c == 0, K0, K1)
        start = jnp.where(c == 0, s * K0, 16 * K0 + s * K1)
        ntriples = nchunks // 3

        pltpu.sync_copy(m_hbm, m_v)

        def head_body(hd, carry):
            pltpu.sync_copy(zer_hbm, numer_s.at[pl.ds(s * NPS, NPS)])
            pltpu.sync_copy(zef_hbm, denom_s.at[pl.ds(s * NPS, NPS)])
            plsc.subcore_barrier()
            mraw = m_v[hd]
            mvec = jnp.where(mraw > 0, mraw, 0.01 * mraw)
            off = hd * N

            def fire(j, b):
                for q in range(CHUNK // 16):
                    w = pck[b, pl.ds(q * 16, 16)]
                    sv = lax.bitwise_and(w, jnp.int32(0xFFFF))
                    dv = lax.shift_right_logical(w, jnp.int32(16))
                    srcc[b, pl.ds(q * 16, 16)] = sv
                    dstc[b, pl.ds(q * 16, 16)] = dv
                    if nh > 1:
                        idxs_v[b, pl.ds(q * 16, 16)] = sv + off
                        idxd_v[b, pl.ds(q * 16, 16)] = dv + off
                if nh > 1:
                    isrc, idst = idxs_v.at[b], idxd_v.at[b]
                else:
                    isrc, idst = srcc.at[b], dstc.at[b]
                pltpu.async_copy(z_hbm.at[isrc], rows_v.at[b], semr[b])
                pltpu.async_copy(sl_hbm.at[isrc], slb.at[b], semsl[b])
                pltpu.async_copy(sr_hbm.at[idst], srb.at[b], semsr[b])

            def proc(j, b):
                isrc = idxs_v.at[b] if nh > 1 else srcc.at[b]
                idst = idxd_v.at[b] if nh > 1 else dstc.at[b]
                pltpu.make_async_copy(sl_hbm.at[isrc], slb.at[b],
                                      semsl[b]).wait()
                pltpu.make_async_copy(sr_hbm.at[idst], srb.at[b],
                                      semsr[b]).wait()
                for q in range(CHUNK // 16):
                    sl16 = slb[b, pl.ds(q * 16, 16)]
                    sr16 = srb[b, pl.ds(q * 16, 16)]
                    e = sl16 + sr16
                    e = jnp.where(e > 0, e, 0.01 * e)
                    ex = jnp.exp(e - mvec)
                    eid = ((start + j) * CHUNK + (q * 16)
                           + lax.iota(jnp.int32, 16))
                    ex = jnp.where(eid < E, ex, 0.0)
                    exv[b, pl.ds(q * 16, 16)] = ex
                pltpu.async_copy(exv.at[b], denom_s.at[dstc.at[b]], semd[b],
                                 add=True)
                pltpu.make_async_copy(z_hbm.at[isrc], rows_v.at[b],
                                      semr[b]).wait()

                def scale_body(g, cc):
                    gs = pl.multiple_of(g * 16, 16)
                    ex16 = exv[b, pl.ds(gs, 16)]
                    for l in range(16):
                        bc = _lane_bcast(ex16, l)
                        r = g * 16 + l
                        for q in range(8):
                            rows_v[b, r, pl.ds(q * 16, 16)] = (
                                rows_v[b, r, pl.ds(q * 16, 16)] * bc)
                    return cc

                lax.fori_loop(0, CHUNK // 16, scale_body, 0)
                pltpu.async_copy(rows_v.at[b], numer_s.at[dstc.at[b]],
                                 semsc[b], add=True)

            def tail(j, b):
                bp = (b + 2) % 3

                def drain_prev():
                    pltpu.make_async_copy(exv.at[bp],
                                          denom_s.at[dstc.at[bp]],
                                          semd[bp]).wait()
                    pltpu.make_async_copy(rows_v.at[bp],
                                          numer_s.at[dstc.at[bp]],
                                          semsc[bp]).wait()

                pl.when(j >= 1)(drain_prev)

                def next_gather():
                    pl.when(j >= 1)(lambda: pltpu.make_async_copy(
                        pk_hbm.at[pl.ds(0, CHUNK)], pck.at[bp],
                        semi[bp]).wait())
                    fire(j + 2, bp)

                pl.when(j + 2 < nchunks)(next_gather)

                def fire_next_idx():
                    pltpu.async_copy(
                        pk_hbm.at[pl.ds((start + j + 3) * CHUNK, CHUNK)],
                        pck.at[b], semi[b])

                pl.when(j + 3 < nchunks)(fire_next_idx)

            for b0 in range(3):
                pltpu.sync_copy(
                    pk_hbm.at[pl.ds((start + b0) * CHUNK, CHUNK)],
                    pck.at[b0])
            fire(0, 0)
            fire(1, 1)

            def triple_body(u, c2):
                for b in range(3):
                    j = 3 * u + b
                    proc(j, b)
                    tail(j, b)
                return c2

            lax.fori_loop(0, ntriples, triple_body, 0)
            pltpu.make_async_copy(exv.at[2], denom_s.at[dstc.at[2]],
                                  semd[2]).wait()
            pltpu.make_async_copy(rows_v.at[2], numer_s.at[dstc.at[2]],
                                  semsc[2]).wait()
            plsc.subcore_barrier()
            pltpu.sync_copy(numer_s.at[pl.ds(s * NPS, NPS)],
                            numer_hbm.at[hd, c, pl.ds(s * NPS, NPS)])
            pltpu.sync_copy(denom_s.at[pl.ds(s * NPS, NPS)],
                            denom_hbm.at[hd, c, pl.ds(s * NPS, NPS)])
            plsc.subcore_barrier()
            return carry

        lax.fori_loop(0, nh, head_body, 0)

    return pl.kernel(
        body,
        compiler_params=pltpu.CompilerParams(needs_layout_passes=False),
        out_type=[
            jax.ShapeDtypeStruct((nh, 2, NP, D), _f32),
            jax.ShapeDtypeStruct((nh, 2, NP), _f32),
        ],
        mesh=plsc.VectorSubcoreMesh(core_axis_name="c", subcore_axis_name="s"),
        scratch_types=[
            pltpu.VMEM((3, CHUNK), jnp.int32),
            pltpu.VMEM((3, CHUNK), jnp.int32),
            pltpu.VMEM((3, CHUNK), jnp.int32),
            pltpu.VMEM((3, CHUNK), jnp.int32),
            pltpu.VMEM((3, CHUNK), jnp.int32),
            pltpu.VMEM((3, CHUNK), _f32),
            pltpu.VMEM((3, CHUNK), _f32),
            pltpu.VMEM((3, CHUNK), _f32),
            pltpu.VMEM((3, CHUNK, D), _f32),
            pltpu.VMEM((nh, 16), _f32),
            pltpu.VMEM_SHARED((NP, D), _f32),
            pltpu.VMEM_SHARED((NP,), _f32),
        ] + [pltpu.SemaphoreType.DMA] * 18,
    )


_sc_agg4 = _make_sc_agg(H)
_sc_agg1 = _make_sc_agg(1)


_BN = 1024


def _tc2_body(p0, p1, p2, p3, d0, d1, d2, d3, w_ref, a_ref,
              z2_ref, sl_ref, sr_ref, m_ref):
    i = pl.program_id(0)
    xs = []
    for p_ref, d_ref in ((p0, d0), (p1, d1), (p2, d2), (p3, d3)):
        num = p_ref[0] + p_ref[1]
        den = jnp.maximum(d_ref[0, pl.ds(i * _BN, _BN)]
                          + d_ref[1, pl.ds(i * _BN, _BN)], 1e-16)
        xh = num / den[:, None]
        xs.append(jnp.where(xh > 0, xh, jnp.exp(xh) - 1.0))
    x = jnp.concatenate(xs, axis=1)
    z2 = lax.dot_general(x, w_ref[...], (((1,), (1,)), ((), ())),
                         preferred_element_type=_f32)
    z2_ref[...] = z2
    sl = jnp.sum(z2 * a_ref[0][None, :], axis=1)
    sr = jnp.sum(z2 * a_ref[1][None, :], axis=1)
    sl_ref[0] = sl
    sr_ref[0] = sr
    m = jnp.full((16,), jnp.max(sl) + jnp.max(sr), _f32)

    @pl.when(i == 0)
    def _init_m():
        m_ref[0] = jnp.full((16,), -3e38, _f32)

    m_ref[0] = jnp.maximum(m_ref[0], m)


_tc2 = pl.pallas_call(
    _tc2_body,
    grid=(NP // _BN,),
    in_specs=[pl.BlockSpec((2, _BN, D), lambda i: (0, i, 0))] * H
    + [pl.BlockSpec((2, NP), lambda i: (0, 0))] * H
    + [pl.BlockSpec((D, H * D), lambda i: (0, 0)),
       pl.BlockSpec((2, D), lambda i: (0, 0))],
    out_specs=[pl.BlockSpec((_BN, D), lambda i: (i, 0)),
               pl.BlockSpec((1, _BN), lambda i: (0, i)),
               pl.BlockSpec((1, _BN), lambda i: (0, i)),
               pl.BlockSpec((1, 16), lambda i: (0, 0))],
    out_shape=[jax.ShapeDtypeStruct((NP, D), _f32),
               jax.ShapeDtypeStruct((1, NP), _f32),
               jax.ShapeDtypeStruct((1, NP), _f32),
               jax.ShapeDtypeStruct((1, 16), _f32)],
)


def _tc3_body(p_ref, d_ref, out_ref):
    i = pl.program_id(0)
    num = p_ref[0] + p_ref[1]
    den = jnp.maximum(d_ref[0, pl.ds(i * _BN, _BN)]
                      + d_ref[1, pl.ds(i * _BN, _BN)], 1e-16)
    out_ref[...] = num / den[:, None]


_tc3 = pl.pallas_call(
    _tc3_body,
    grid=(NP // _BN,),
    in_specs=[pl.BlockSpec((2, _BN, D), lambda i: (0, i, 0)),
              pl.BlockSpec((2, NP), lambda i: (0, 0))],
    out_specs=pl.BlockSpec((_BN, D), lambda i: (i, 0)),
    out_shape=jax.ShapeDtypeStruct((NP, D), _f32),
)


@jax.jit
def kernel(h, edge_index, W1, a1, W2, a2):
    zer = jnp.zeros((NPS, D), _f32)
    zef = jnp.zeros((NPS,), _f32)
    Z1, SL1, SR1, M1, pkj2 = _tc1(h, W1, a1[:, None, :], edge_index)
    pkj = pkj2[0]

    numer1, denom1 = _sc_agg4(Z1.reshape(H * N, D), SL1.reshape(H * N),
                              SR1.reshape(H * N), M1.reshape(H, 16),
                              pkj, zer, zef)

    z2, sl2, sr2, m2 = _tc2(numer1[0], numer1[1], numer1[2], numer1[3],
                            denom1[0], denom1[1], denom1[2], denom1[3],
                            W2, a2.reshape(2, D))
    numer2, denom2 = _sc_agg1(z2, sl2[0], sr2[0], m2, pkj, zer, zef)
    return _tc3(numer2[0], denom2[0])[:N]
